# Initial kernel scaffold; baseline (speedup 1.0000x reference)
import jax
import jax.numpy as jnp
from jax.experimental import pallas as pl


def kernel(w0, b0, w1, b1, w2, b2, w_fc1, b_fc1, w_fc2, b_fc2, x):
    raise NotImplementedError("write your pallas kernel here")


# speedup vs baseline: 4.0209x; 4.0209x over previous
import jax
import jax.numpy as jnp
from jax.experimental import pallas as pl
from jax.experimental.pallas import tpu as pltpu

K5 = 5
TAPS = K5 * K5
PACK = 4
ROWPAD = 8


def _conv_taps(x_ref, w_ref, b_ref, out_ref, *, ws, m):
    lout = out_ref.shape[-1]
    acc = jnp.zeros((m, lout), jnp.float32)
    for kh in range(K5):
        for kw in range(K5):
            off = kh * ws + kw
            acc += jnp.dot(x_ref[off:off + m, :], w_ref[kh * K5 + kw],
                           preferred_element_type=jnp.float32)
    out_ref[0:m, :] = jnp.maximum(acc + b_ref[...], 0.0).astype(out_ref.dtype)
    pad = out_ref.shape[0] - m
    if pad:
        out_ref[m:, :] = jnp.zeros((pad, lout), out_ref.dtype)


def _pool2x2(x_ref, out_ref, *, h, ws):
    lanes = out_ref.shape[-1]
    h2, w2 = h // 2, ws // 2
    v = x_ref[0:h * ws, :].reshape(h2, 2, w2, 2, lanes)
    pooled = jnp.max(jnp.max(v, axis=3), axis=1).reshape(h2 * w2, lanes)
    out_ref[0:h2 * w2, :] = pooled
    pad = out_ref.shape[0] - h2 * w2
    if pad:
        out_ref[h2 * w2:, :] = jnp.zeros((pad, lanes), out_ref.dtype)


def _trunk_kernel(x_ref, w0_ref, b0_ref, w1_ref, b1_ref, w2_ref, b2_ref,
                  o_ref, s0, s1, s1p, s2, s2p):
    _conv_taps(x_ref, w0_ref, b0_ref, s0, ws=32, m=28 * 32)
    _conv_taps(s0, w1_ref, b1_ref, s1, ws=32, m=24 * 32)
    _pool2x2(s1, s1p, h=24, ws=32)
    _conv_taps(s1p, w2_ref, b2_ref, s2, ws=16, m=8 * 16)
    _pool2x2(s2, s2p, h=8, ws=16)
    for j in range(PACK):
        o_ref[j, :, :] = s2p[:, 64 * j:64 * (j + 1)]


def _conv_trunk(x4, w0, b0, w1, b1, w2, b2):
    g = x4.shape[0]
    bf = jnp.bfloat16
    return pl.pallas_call(
        _trunk_kernel,
        grid=(g,),
        in_specs=[
            pl.BlockSpec((None, 1032, 32), lambda i: (i, 0, 0)),
            pl.BlockSpec((TAPS, 32, 128), lambda i: (0, 0, 0)),
            pl.BlockSpec((1, 128), lambda i: (0, 0)),
            pl.BlockSpec((TAPS, 128, 128), lambda i: (0, 0, 0)),
            pl.BlockSpec((1, 128), lambda i: (0, 0)),
            pl.BlockSpec((TAPS, 128, 256), lambda i: (0, 0, 0)),
            pl.BlockSpec((1, 256), lambda i: (0, 0)),
        ],
        out_specs=pl.BlockSpec((PACK, 32, 64), lambda i: (i, 0, 0)),
        out_shape=jax.ShapeDtypeStruct((g * PACK, 32, 64), bf),
        scratch_shapes=[
            pltpu.VMEM((28 * 32 + ROWPAD, 128), bf),
            pltpu.VMEM((24 * 32, 128), bf),
            pltpu.VMEM((12 * 16 + ROWPAD, 128), bf),
            pltpu.VMEM((8 * 16, 256), bf),
            pltpu.VMEM((4 * 8, 256), bf),
        ],
        compiler_params=pltpu.CompilerParams(
            dimension_semantics=("parallel",),
            vmem_limit_bytes=64 * 1024 * 1024),
    )(x4, w0, b0, w1, b1, w2, b2)


def _classifier_kernel(a_ref, w1_ref, b1_ref, w2_ref, b2_ref, o_ref):
    h = jnp.maximum(jnp.dot(a_ref[...], w1_ref[...],
                            preferred_element_type=jnp.float32)
                    + b1_ref[...], 0.0).astype(jnp.bfloat16)
    o_ref[...] = jnp.dot(h, w2_ref[...],
                         preferred_element_type=jnp.float32) + b2_ref[...]


def _classifier(a, w1, b1, w2, b2):
    m, kdim = a.shape
    n1, n2 = w1.shape[1], w2.shape[1]
    tm = 512 if m % 512 == 0 else m
    return pl.pallas_call(
        _classifier_kernel,
        grid=(m // tm,),
        in_specs=[
            pl.BlockSpec((tm, kdim), lambda i: (i, 0)),
            pl.BlockSpec((kdim, n1), lambda i: (0, 0)),
            pl.BlockSpec((1, n1), lambda i: (0, 0)),
            pl.BlockSpec((n1, n2), lambda i: (0, 0)),
            pl.BlockSpec((1, n2), lambda i: (0, 0)),
        ],
        out_specs=pl.BlockSpec((tm, n2), lambda i: (i, 0)),
        out_shape=jax.ShapeDtypeStruct((m, n2), jnp.float32),
        compiler_params=pltpu.CompilerParams(
            dimension_semantics=("parallel",)),
    )(a, w1, b1, w2, b2)


def _block_diag4(w):
    eye = jnp.eye(PACK, dtype=jnp.float32)
    t, r, c = w.shape
    out = jnp.einsum("ij,tkl->tikjl", eye, w).reshape(t, PACK * r, PACK * c)
    return out.astype(jnp.bfloat16)


def kernel(w0, b0, w1, b1, w2, b2, w_fc1, b_fc1, w_fc2, b_fc2, x):
    n = x.shape[0]
    g = n // PACK

    w0p = _block_diag4(w0[:, :8, :32])
    w1p = _block_diag4(w1[:, :32, :32])
    w2p = _block_diag4(w2[:, :32, :64])
    b0p = jnp.tile(b0[:, :32], (1, PACK))
    b1p = jnp.tile(b1[:, :32], (1, PACK))
    b2p = jnp.tile(b2[:, :64], (1, PACK))
    wf1 = w_fc1.reshape(32, 128, 512)[:, :64, :].reshape(2048, 512)
    wf1 = wf1.astype(jnp.bfloat16)
    wf2 = w_fc2.astype(jnp.bfloat16)

    xb = x.astype(jnp.bfloat16)
    xt = jnp.transpose(xb, (0, 2, 3, 1))
    xt = jnp.pad(xt, ((0, 0), (0, 0), (0, 0), (0, 5)))
    xt = xt.reshape(n, 1024, 8)
    xt = jnp.pad(xt, ((0, 0), (0, ROWPAD), (0, 0)))
    x4 = xt.reshape(g, PACK, 1032, 8).transpose(0, 2, 1, 3).reshape(g, 1032, 32)

    feat = _conv_trunk(x4, w0p, b0p, w1p, b1p, w2p, b2p)
    logits = _classifier(feat.reshape(n, 2048), wf1, b_fc1, wf2, b_fc2)
    return logits[:, :10]


# speedup vs baseline: 4.4647x; 1.1104x over previous
import jax
import jax.numpy as jnp
from jax.experimental import pallas as pl
from jax.experimental.pallas import tpu as pltpu

K5 = 5
TAPS = K5 * K5
PACK = 4
ROWPAD = 8


def _conv_taps(x_ref, xs_ref, w_ref, b_ref, out_ref, *, ws, m):
    rows = xs_ref.shape[1]
    for kw in range(K5):
        xs_ref[kw, :, :] = x_ref[kw:kw + rows, :]
    lout = out_ref.shape[-1]
    acc = jnp.zeros((m, lout), jnp.float32)
    for kh in range(K5):
        for kw in range(K5):
            acc += jnp.dot(xs_ref[kw, kh * ws:kh * ws + m, :],
                           w_ref[kh * K5 + kw],
                           preferred_element_type=jnp.float32)
    out_ref[0:m, :] = jnp.maximum(acc + b_ref[...], 0.0).astype(out_ref.dtype)
    pad = out_ref.shape[0] - m
    if pad:
        out_ref[m:, :] = jnp.zeros((pad, lout), out_ref.dtype)


def _pool2x2(x_ref, s_ref, *, h, ws):
    lanes = x_ref.shape[-1]
    h2 = h // 2
    v = x_ref[0:h * ws, :].reshape(h2, 2 * ws, lanes)
    a = jnp.maximum(v[:, :ws, :], v[:, ws:, :]).reshape(h2 * ws, lanes)
    shifted = jnp.concatenate(
        [a[1:, :], jnp.zeros((1, lanes), a.dtype)], axis=0)
    c = jnp.maximum(a, shifted)
    return jnp.dot(s_ref[...], c,
                   preferred_element_type=jnp.float32).astype(x_ref.dtype)


def _trunk_kernel(x_ref, x5_ref, w00_ref, w0c_ref, b0_ref, w1_ref, b1_ref,
                  w2_ref, b2_ref, s1_sel, s2_sel,
                  o_ref, s0, s1, s1p, s2, xs1, xs2):
    m0 = 28 * 32
    acc = jnp.zeros((m0, 128), jnp.float32)
    for kh in range(K5):
        r = kh * 32
        acc += jnp.dot(x_ref[r:r + m0, :], w00_ref[kh],
                       preferred_element_type=jnp.float32)
        acc += jnp.dot(x5_ref[r:r + m0, :], w0c_ref[kh],
                       preferred_element_type=jnp.float32)
    s0[0:m0, :] = jnp.maximum(acc + b0_ref[...], 0.0).astype(s0.dtype)
    s0[m0:, :] = jnp.zeros((ROWPAD, 128), s0.dtype)
    _conv_taps(s0, xs1, w1_ref, b1_ref, s1, ws=32, m=24 * 32)
    s1p[0:192, :] = _pool2x2(s1, s1_sel, h=24, ws=32)
    s1p[192:, :] = jnp.zeros((ROWPAD, 128), s1p.dtype)
    _conv_taps(s1p, xs2, w2_ref, b2_ref, s2, ws=16, m=8 * 16)
    pooled = _pool2x2(s2, s2_sel, h=8, ws=16)
    for j in range(PACK):
        o_ref[j, :, :] = pooled[:, 64 * j:64 * (j + 1)]


def _conv_trunk(x4, x5, w00, w0c, b0, w1, b1, w2, b2, s1_sel, s2_sel):
    g = x4.shape[0]
    bf = jnp.bfloat16
    return pl.pallas_call(
        _trunk_kernel,
        grid=(g,),
        in_specs=[
            pl.BlockSpec((None, 1032, 32), lambda i: (i, 0, 0)),
            pl.BlockSpec((None, 1032, 128), lambda i: (i, 0, 0)),
            pl.BlockSpec((K5, 32, 128), lambda i: (0, 0, 0)),
            pl.BlockSpec((K5, 128, 128), lambda i: (0, 0, 0)),
            pl.BlockSpec((1, 128), lambda i: (0, 0)),
            pl.BlockSpec((TAPS, 128, 128), lambda i: (0, 0, 0)),
            pl.BlockSpec((1, 128), lambda i: (0, 0)),
            pl.BlockSpec((TAPS, 128, 256), lambda i: (0, 0, 0)),
            pl.BlockSpec((1, 256), lambda i: (0, 0)),
            pl.BlockSpec((192, 384), lambda i: (0, 0)),
            pl.BlockSpec((32, 64), lambda i: (0, 0)),
        ],
        out_specs=pl.BlockSpec((PACK, 32, 64), lambda i: (i, 0, 0)),
        out_shape=jax.ShapeDtypeStruct((g * PACK, 32, 64), bf),
        scratch_shapes=[
            pltpu.VMEM((28 * 32 + ROWPAD, 128), bf),
            pltpu.VMEM((24 * 32, 128), bf),
            pltpu.VMEM((12 * 16 + ROWPAD, 128), bf),
            pltpu.VMEM((8 * 16, 256), bf),
            pltpu.VMEM((K5, 900, 128), bf),
            pltpu.VMEM((K5, 196, 128), bf),
        ],
        compiler_params=pltpu.CompilerParams(
            dimension_semantics=("parallel",),
            vmem_limit_bytes=64 * 1024 * 1024),
    )(x4, x5, w00, w0c, b0, w1, b1, w2, b2, s1_sel, s2_sel)


def _classifier_kernel(a_ref, w1_ref, b1_ref, w2_ref, b2_ref, o_ref):
    h = jnp.maximum(jnp.dot(a_ref[...], w1_ref[...],
                            preferred_element_type=jnp.float32)
                    + b1_ref[...], 0.0).astype(jnp.bfloat16)
    o_ref[...] = jnp.dot(h, w2_ref[...],
                         preferred_element_type=jnp.float32) + b2_ref[...]


def _classifier(a, w1, b1, w2, b2):
    m, kdim = a.shape
    n1, n2 = w1.shape[1], w2.shape[1]
    tm = 512 if m % 512 == 0 else m
    return pl.pallas_call(
        _classifier_kernel,
        grid=(m // tm,),
        in_specs=[
            pl.BlockSpec((tm, kdim), lambda i: (i, 0)),
            pl.BlockSpec((kdim, n1), lambda i: (0, 0)),
            pl.BlockSpec((1, n1), lambda i: (0, 0)),
            pl.BlockSpec((n1, n2), lambda i: (0, 0)),
            pl.BlockSpec((1, n2), lambda i: (0, 0)),
        ],
        out_specs=pl.BlockSpec((tm, n2), lambda i: (i, 0)),
        out_shape=jax.ShapeDtypeStruct((m, n2), jnp.float32),
        compiler_params=pltpu.CompilerParams(
            dimension_semantics=("parallel",)),
    )(a, w1, b1, w2, b2)


def _block_diag4(w):
    eye = jnp.eye(PACK, dtype=jnp.float32)
    t, r, c = w.shape
    out = jnp.einsum("ij,tkl->tikjl", eye, w).reshape(t, PACK * r, PACK * c)
    return out.astype(jnp.bfloat16)


def kernel(w0, b0, w1, b1, w2, b2, w_fc1, b_fc1, w_fc2, b_fc2, x):
    n = x.shape[0]
    g = n // PACK

    w0p = _block_diag4(w0[:, :8, :32])
    w1p = _block_diag4(w1[:, :32, :32])
    w2p = _block_diag4(w2[:, :32, :64])
    b0p = jnp.tile(b0[:, :32], (1, PACK))
    b1p = jnp.tile(b1[:, :32], (1, PACK))
    b2p = jnp.tile(b2[:, :64], (1, PACK))
    w0r = w0p.reshape(K5, K5, 32, 128)
    w00 = w0r[:, 0]
    w0c = w0r[:, 1:].reshape(K5, 128, 128)
    hh1, ww1 = jnp.meshgrid(jnp.arange(12), jnp.arange(16), indexing="ij")
    s1_sel = jnp.zeros((192, 384), jnp.float32).at[
        (hh1 * 16 + ww1).ravel(), (hh1 * 32 + 2 * ww1).ravel()].set(1.0)
    hh2, ww2 = jnp.meshgrid(jnp.arange(4), jnp.arange(8), indexing="ij")
    s2_sel = jnp.zeros((32, 64), jnp.float32).at[
        (hh2 * 8 + ww2).ravel(), (hh2 * 16 + 2 * ww2).ravel()].set(1.0)
    s1_sel = s1_sel.astype(jnp.bfloat16)
    s2_sel = s2_sel.astype(jnp.bfloat16)
    wf1 = w_fc1.reshape(32, 128, 512)[:, :64, :].reshape(2048, 512)
    wf1 = wf1.astype(jnp.bfloat16)
    wf2 = w_fc2.astype(jnp.bfloat16)

    xb = x.astype(jnp.bfloat16)
    xt = jnp.transpose(xb, (0, 2, 3, 1))
    xt = jnp.pad(xt, ((0, 0), (0, 0), (0, 0), (0, 5)))
    xt = xt.reshape(n, 1024, 8)
    xt = jnp.pad(xt, ((0, 0), (0, ROWPAD), (0, 0)))
    x4 = xt.reshape(g, PACK, 1032, 8).transpose(0, 2, 1, 3).reshape(g, 1032, 32)
    x5 = jnp.concatenate(
        [jnp.pad(x4[:, s:, :], ((0, 0), (0, s), (0, 0))) for s in (1, 2, 3, 4)],
        axis=2)

    feat = _conv_trunk(x4, x5, w00, w0c, b0p, w1p, b1p, w2p, b2p,
                       s1_sel, s2_sel)
    logits = _classifier(feat.reshape(n, 2048), wf1, b_fc1, wf2, b_fc2)
    return logits[:, :10]
